# baseline (speedup 1.0000x reference)
import functools

import jax
import jax.numpy as jnp
import numpy as np
from jax import lax
from jax.experimental import pallas as pl
from jax.experimental.pallas import tpu as pltpu
from jax.experimental.pallas import tpu_sc as plsc

F = 1025
T = 256
B = 2
MAXP = 5
MARGIN = 3
PWR = 0.1
LLIM = F - (MARGIN + 1)


def _tc_body(x_ref, o_ref):
    a = x_ref[:, 1:, :]
    rows = lax.broadcasted_iota(jnp.int32, a.shape, 1)
    work = a
    theta = None
    for _ in range(MAXP):
        mj = jnp.max(work, axis=1, keepdims=True)
        hit = work == mj
        r = jnp.min(jnp.where(hit, rows, F), axis=1, keepdims=True)
        work = jnp.where(rows == r, -jnp.inf, work)
        theta = mj
    ok = (a >= theta) & (a > PWR)
    f0 = jnp.min(jnp.where(ok, rows + 1, F), axis=1, keepdims=True)
    f0 = jnp.where(f0 == F, 0, f0)
    f0f = f0.astype(jnp.float32)
    safe = jnp.maximum(f0f, 1.0)
    kk = lax.broadcasted_iota(jnp.int32, (B, F, T), 1).astype(jnp.float32)
    mmax = jnp.floor(jnp.float32(LLIM - 1) / safe)
    m = jnp.minimum(mmax, jnp.floor((kk + MARGIN) / safe))
    d = jnp.abs(kk - m * f0f)
    cover = (f0f > 0.0) & (m >= 1.0) & (d <= MARGIN)
    val = jnp.maximum(1.0 - d * (0.5 / MARGIN), 0.5)
    o_ref[...] = jnp.where(cover, val, jnp.float32(0.5))


@functools.partial(jax.jit, static_argnames=("interpret",))
def _tc_mask(x2, interpret=False):
    return pl.pallas_call(
        _tc_body,
        out_shape=jax.ShapeDtypeStruct((B, F, T), jnp.float32),
        interpret=interpret,
    )(x2)


LANES = 16


def _bubble(ms, v):
    out = []
    for mj in ms:
        out.append(jnp.maximum(mj, v))
        v = jnp.minimum(mj, v)
    return tuple(out)


CHUNK = 16
NCH = (F - 1) // CHUNK
PWR_NEXT = float(np.nextafter(np.float32(PWR), np.float32(1.0)))


def _sc_body(x_hbm, o_hbm, in_v, cm_v, sem):
    b = lax.axis_index("c")
    t0 = lax.axis_index("s") * LANES
    pltpu.async_copy(x_hbm.at[b, 0, :, pl.ds(t0, LANES)], in_v, sem).wait()
    lane = lax.broadcasted_iota(jnp.int32, (LANES,), 0)
    neg = jnp.full((LANES,), -jnp.inf, jnp.float32)
    zero = jnp.zeros((LANES,), jnp.int32)

    def pa(bi, _):
        base = 1 + bi * CHUNK
        m = in_v[base]
        for r in range(1, CHUNK):
            m = jnp.maximum(m, in_v[base + r])
        cm_v[bi] = m
        return 0

    lax.fori_loop(0, NCH, pa, 0, unroll=2)

    def pb(bi, carry):
        vals, idxs = carry
        v = cm_v[bi]
        iv = jnp.full((LANES,), bi, jnp.int32)
        nv, ni = [], []
        for mj, ij in zip(vals, idxs):
            c = v > mj
            nv.append(jnp.where(c, v, mj))
            ni.append(jnp.where(c, iv, ij))
            v, iv = jnp.where(c, mj, v), jnp.where(c, ij, iv)
        return tuple(nv), tuple(ni)

    _, cidxs = lax.fori_loop(0, NCH, pb, ((neg,) * MAXP, (zero,) * MAXP),
                             unroll=2)

    ms = (neg,) * MAXP
    for j in range(MAXP):
        base = 1 + cidxs[j] * CHUNK
        for r in range(CHUNK):
            ms = _bubble(ms, plsc.load_gather(in_v, [base + r, lane]))
    thr = jnp.maximum(ms[MAXP - 1], jnp.float32(PWR_NEXT))

    def p2(bi, bf):
        c = cm_v[bi] >= thr
        return jnp.minimum(bf, jnp.where(c, jnp.full((LANES,), bi), NCH))

    bfirst = lax.fori_loop(0, NCH, p2, jnp.full((LANES,), NCH, jnp.int32),
                           unroll=4)
    base = 1 + jnp.minimum(bfirst, NCH - 1) * CHUNK
    best = jnp.full((LANES,), F, jnp.int32)
    for r in range(CHUNK):
        g = plsc.load_gather(in_v, [base + r, lane])
        best = jnp.minimum(best, jnp.where(g >= thr, base + r, F))
    f0i = jnp.where(bfirst == NCH, 0, best)

    si = jnp.maximum(f0i, 1)
    mmax = (jnp.float32(LLIM - 1) / si.astype(jnp.float32)).astype(jnp.int32)
    mmax = jnp.where((mmax + 1) * si <= LLIM - 1, mmax + 1, mmax)
    mmax = jnp.where(mmax * si > LLIM - 1, mmax - 1, mmax)
    covered = (f0i > 0) & (mmax >= 1)
    ncmax = mmax * f0i
    mi = jnp.where(f0i == 1, 2, jnp.where(f0i == 2, 1, 0))
    c0 = jnp.maximum(mi, 1) * f0i
    nc0 = (mi + 1) * f0i

    def p3(k, carry):
        kv, kk3, c, nc = carry
        bump = (kk3 >= nc) & (nc <= ncmax)
        c = jnp.where(bump, nc, c)
        nc = jnp.where(bump, nc + f0i, nc)
        d = jnp.abs(kv - c)
        d = jnp.where(covered, d, MARGIN + 1)
        val = jnp.maximum(1.0 - d.astype(jnp.float32) * (0.5 / MARGIN), 0.5)
        in_v[k] = val
        return kv + 1, kk3 + 1, c, nc

    three = jnp.full((LANES,), MARGIN, jnp.int32)
    lax.fori_loop(0, F, p3, (zero, three, c0, nc0), unroll=8)

    pltpu.async_copy(in_v, o_hbm.at[b, 0, :, pl.ds(t0, LANES)], sem).wait()


@jax.jit
def _sc_mask(x2):
    kern = pl.kernel(
        _sc_body,
        out_type=jax.ShapeDtypeStruct((B, 1, F, T), jnp.float32),
        mesh=plsc.VectorSubcoreMesh(core_axis_name="c", subcore_axis_name="s"),
        compiler_params=pltpu.CompilerParams(
            use_tc_tiling_on_sc=False, needs_layout_passes=False),
        scratch_types=[
            pltpu.VMEM((F, LANES), jnp.float32),
            pltpu.VMEM((NCH, LANES), jnp.float32),
            pltpu.SemaphoreType.DMA,
        ],
    )
    return kern(x2)


def kernel(x):
    xf = lax.optimization_barrier(x.reshape(-1))
    out = _sc_mask(xf.reshape(B, 1, F, T))
    of = lax.optimization_barrier(out.reshape(-1))
    return of.reshape(B, 1, F, T)


# speedup vs baseline: 1.2823x; 1.2823x over previous
import functools

import jax
import jax.numpy as jnp
import numpy as np
from jax import lax
from jax.experimental import pallas as pl
from jax.experimental.pallas import tpu as pltpu
from jax.experimental.pallas import tpu_sc as plsc

F = 1025
T = 256
B = 2
MAXP = 5
MARGIN = 3
PWR = 0.1
LLIM = F - (MARGIN + 1)


def _tc_body(x_ref, o_ref):
    a = x_ref[:, 1:, :]
    rows = lax.broadcasted_iota(jnp.int32, a.shape, 1)
    work = a
    theta = None
    for _ in range(MAXP):
        mj = jnp.max(work, axis=1, keepdims=True)
        hit = work == mj
        r = jnp.min(jnp.where(hit, rows, F), axis=1, keepdims=True)
        work = jnp.where(rows == r, -jnp.inf, work)
        theta = mj
    ok = (a >= theta) & (a > PWR)
    f0 = jnp.min(jnp.where(ok, rows + 1, F), axis=1, keepdims=True)
    f0 = jnp.where(f0 == F, 0, f0)
    f0f = f0.astype(jnp.float32)
    safe = jnp.maximum(f0f, 1.0)
    kk = lax.broadcasted_iota(jnp.int32, (B, F, T), 1).astype(jnp.float32)
    mmax = jnp.floor(jnp.float32(LLIM - 1) / safe)
    m = jnp.minimum(mmax, jnp.floor((kk + MARGIN) / safe))
    d = jnp.abs(kk - m * f0f)
    cover = (f0f > 0.0) & (m >= 1.0) & (d <= MARGIN)
    val = jnp.maximum(1.0 - d * (0.5 / MARGIN), 0.5)
    o_ref[...] = jnp.where(cover, val, jnp.float32(0.5))


@functools.partial(jax.jit, static_argnames=("interpret",))
def _tc_mask(x2, interpret=False):
    return pl.pallas_call(
        _tc_body,
        out_shape=jax.ShapeDtypeStruct((B, F, T), jnp.float32),
        interpret=interpret,
    )(x2)


LANES = 16
FP = 1032


def _stage_body(x_ref, o_ref):
    o_ref[:, :F, :] = x_ref[...]
    o_ref[:, F:, :] = jnp.full((B, FP - F, T), 0.5, jnp.float32)


@jax.jit
def _tc_stage(x3):
    return pl.pallas_call(
        _stage_body,
        out_shape=jax.ShapeDtypeStruct((B, FP, T), jnp.float32),
    )(x3)


def _bubble(ms, v):
    out = []
    for mj in ms:
        out.append(jnp.maximum(mj, v))
        v = jnp.minimum(mj, v)
    return tuple(out)


CHUNK = 16
NCH = (F - 1) // CHUNK
PWR_NEXT = float(np.nextafter(np.float32(PWR), np.float32(1.0)))


def _sc_body(x_hbm, o_hbm, in_v, cm_v, sem):
    b = lax.axis_index("c")
    t0 = lax.axis_index("s") * LANES
    pltpu.async_copy(x_hbm.at[b, pl.ds(0, F), pl.ds(t0, LANES)], in_v,
                     sem).wait()
    lane = lax.broadcasted_iota(jnp.int32, (LANES,), 0)
    neg = jnp.full((LANES,), -jnp.inf, jnp.float32)
    zero = jnp.zeros((LANES,), jnp.int32)

    def pa(bi, _):
        base = 1 + bi * CHUNK
        m = in_v[base]
        for r in range(1, CHUNK):
            m = jnp.maximum(m, in_v[base + r])
        cm_v[bi] = m
        return 0

    lax.fori_loop(0, NCH, pa, 0, unroll=2)

    def pb(bi, carry):
        vals, idxs = carry
        v = cm_v[bi]
        iv = jnp.full((LANES,), bi, jnp.int32)
        nv, ni = [], []
        for mj, ij in zip(vals, idxs):
            c = v > mj
            nv.append(jnp.where(c, v, mj))
            ni.append(jnp.where(c, iv, ij))
            v, iv = jnp.where(c, mj, v), jnp.where(c, ij, iv)
        return tuple(nv), tuple(ni)

    _, cidxs = lax.fori_loop(0, NCH, pb, ((neg,) * MAXP, (zero,) * MAXP),
                             unroll=2)

    ms = (neg,) * MAXP
    for j in range(MAXP):
        base = 1 + cidxs[j] * CHUNK
        for r in range(CHUNK):
            ms = _bubble(ms, plsc.load_gather(in_v, [base + r, lane]))
    thr = jnp.maximum(ms[MAXP - 1], jnp.float32(PWR_NEXT))

    def p2(bi, bf):
        c = cm_v[bi] >= thr
        return jnp.minimum(bf, jnp.where(c, jnp.full((LANES,), bi), NCH))

    bfirst = lax.fori_loop(0, NCH, p2, jnp.full((LANES,), NCH, jnp.int32),
                           unroll=4)
    base = 1 + jnp.minimum(bfirst, NCH - 1) * CHUNK
    best = jnp.full((LANES,), F, jnp.int32)
    for r in range(CHUNK):
        g = plsc.load_gather(in_v, [base + r, lane])
        best = jnp.minimum(best, jnp.where(g >= thr, base + r, F))
    f0i = jnp.where(bfirst == NCH, 0, best)

    si = jnp.maximum(f0i, 1)
    mmax = (jnp.float32(LLIM - 1) / si.astype(jnp.float32)).astype(jnp.int32)
    mmax = jnp.where((mmax + 1) * si <= LLIM - 1, mmax + 1, mmax)
    mmax = jnp.where(mmax * si > LLIM - 1, mmax - 1, mmax)
    covered = (f0i > 0) & (mmax >= 1)
    ncmax = mmax * f0i
    mi = jnp.where(f0i == 1, 2, jnp.where(f0i == 2, 1, 0))
    c0 = jnp.maximum(mi, 1) * f0i
    nc0 = (mi + 1) * f0i

    def p3(k, carry):
        kv, kk3, c, nc = carry
        bump = (kk3 >= nc) & (nc <= ncmax)
        c = jnp.where(bump, nc, c)
        nc = jnp.where(bump, nc + f0i, nc)
        d = jnp.abs(kv - c)
        d = jnp.where(covered, d, MARGIN + 1)
        val = jnp.maximum(1.0 - d.astype(jnp.float32) * (0.5 / MARGIN), 0.5)
        in_v[k] = val
        return kv + 1, kk3 + 1, c, nc

    three = jnp.full((LANES,), MARGIN, jnp.int32)
    lax.fori_loop(0, F, p3, (zero, three, c0, nc0), unroll=8)

    pltpu.async_copy(in_v, o_hbm.at[b, 0, pl.ds(0, F), pl.ds(t0, LANES)],
                     sem).wait()


@jax.jit
def _sc_mask(x2):
    kern = pl.kernel(
        _sc_body,
        out_type=jax.ShapeDtypeStruct((B, 1, F, T), jnp.float32),
        mesh=plsc.VectorSubcoreMesh(core_axis_name="c", subcore_axis_name="s"),
        compiler_params=pltpu.CompilerParams(
            use_tc_tiling_on_sc=False, needs_layout_passes=False),
        scratch_types=[
            pltpu.VMEM((F, LANES), jnp.float32),
            pltpu.VMEM((NCH, LANES), jnp.float32),
            pltpu.SemaphoreType.DMA,
        ],
    )
    return kern(x2)


def kernel(x):
    xp = _tc_stage(x.reshape(B, F, T))
    return _sc_mask(xp)


# speedup vs baseline: 1.4834x; 1.1569x over previous
import functools

import jax
import jax.numpy as jnp
import numpy as np
from jax import lax
from jax.experimental import pallas as pl
from jax.experimental.pallas import tpu as pltpu
from jax.experimental.pallas import tpu_sc as plsc

F = 1025
T = 256
B = 2
MAXP = 5
MARGIN = 3
PWR = 0.1
LLIM = F - (MARGIN + 1)


def _tc_body(x_ref, o_ref):
    a = x_ref[:, 1:, :]
    rows = lax.broadcasted_iota(jnp.int32, a.shape, 1)
    work = a
    theta = None
    for _ in range(MAXP):
        mj = jnp.max(work, axis=1, keepdims=True)
        hit = work == mj
        r = jnp.min(jnp.where(hit, rows, F), axis=1, keepdims=True)
        work = jnp.where(rows == r, -jnp.inf, work)
        theta = mj
    ok = (a >= theta) & (a > PWR)
    f0 = jnp.min(jnp.where(ok, rows + 1, F), axis=1, keepdims=True)
    f0 = jnp.where(f0 == F, 0, f0)
    f0f = f0.astype(jnp.float32)
    safe = jnp.maximum(f0f, 1.0)
    kk = lax.broadcasted_iota(jnp.int32, (B, F, T), 1).astype(jnp.float32)
    mmax = jnp.floor(jnp.float32(LLIM - 1) / safe)
    m = jnp.minimum(mmax, jnp.floor((kk + MARGIN) / safe))
    d = jnp.abs(kk - m * f0f)
    cover = (f0f > 0.0) & (m >= 1.0) & (d <= MARGIN)
    val = jnp.maximum(1.0 - d * (0.5 / MARGIN), 0.5)
    o_ref[...] = jnp.where(cover, val, jnp.float32(0.5))


@functools.partial(jax.jit, static_argnames=("interpret",))
def _tc_mask(x2, interpret=False):
    return pl.pallas_call(
        _tc_body,
        out_shape=jax.ShapeDtypeStruct((B, F, T), jnp.float32),
        interpret=interpret,
    )(x2)


LANES = 16
FP = 1032


def _stage_body(x_ref, o_ref):
    o_ref[:, :F, :] = x_ref[...]
    o_ref[:, F:, :] = jnp.full((B, FP - F, T), 0.5, jnp.float32)


@jax.jit
def _tc_stage(x3):
    return pl.pallas_call(
        _stage_body,
        out_shape=jax.ShapeDtypeStruct((B, FP, T), jnp.float32),
    )(x3)


def _bubble(ms, v):
    out = []
    for mj in ms:
        out.append(jnp.maximum(mj, v))
        v = jnp.minimum(mj, v)
    return tuple(out)


CHUNK = 16
NCH = (F - 1) // CHUNK
PWR_NEXT = float(np.nextafter(np.float32(PWR), np.float32(1.0)))


def _sc_body(x_hbm, o_hbm, in_v, cm_v, sem):
    b = lax.axis_index("c")
    t0 = lax.axis_index("s") * LANES
    pltpu.async_copy(x_hbm.at[b, pl.ds(0, F), pl.ds(t0, LANES)], in_v,
                     sem).wait()
    lane = lax.broadcasted_iota(jnp.int32, (LANES,), 0)
    neg = jnp.full((LANES,), -jnp.inf, jnp.float32)
    zero = jnp.zeros((LANES,), jnp.int32)

    def pa(bi, _):
        base = 1 + bi * CHUNK
        m = in_v[base]
        for r in range(1, CHUNK):
            m = jnp.maximum(m, in_v[base + r])
        cm_v[bi] = m
        return 0

    lax.fori_loop(0, NCH, pa, 0, unroll=2)

    def pb(bi, carry):
        vals, idxs = carry
        v = cm_v[bi]
        iv = jnp.full((LANES,), bi, jnp.int32)
        nv, ni = [], []
        for mj, ij in zip(vals, idxs):
            c = v > mj
            nv.append(jnp.where(c, v, mj))
            ni.append(jnp.where(c, iv, ij))
            v, iv = jnp.where(c, mj, v), jnp.where(c, ij, iv)
        return tuple(nv), tuple(ni)

    _, cidxs = lax.fori_loop(0, NCH, pb, ((neg,) * MAXP, (zero,) * MAXP),
                             unroll=2)

    ms = (neg,) * MAXP
    for j in range(MAXP):
        base = 1 + cidxs[j] * CHUNK
        for r in range(CHUNK):
            ms = _bubble(ms, plsc.load_gather(in_v, [base + r, lane]))
    thr = jnp.maximum(ms[MAXP - 1], jnp.float32(PWR_NEXT))

    def p2(bi, bf):
        c = cm_v[bi] >= thr
        return jnp.minimum(bf, jnp.where(c, jnp.full((LANES,), bi), NCH))

    bfirst = lax.fori_loop(0, NCH, p2, jnp.full((LANES,), NCH, jnp.int32),
                           unroll=4)
    base = 1 + jnp.minimum(bfirst, NCH - 1) * CHUNK
    best = jnp.full((LANES,), F, jnp.int32)
    for r in range(CHUNK):
        g = plsc.load_gather(in_v, [base + r, lane])
        best = jnp.minimum(best, jnp.where(g >= thr, base + r, F))
    f0i = jnp.where(bfirst == NCH, 0, best)

    si = jnp.maximum(f0i, 1)
    mmax = (jnp.float32(LLIM - 1) / si.astype(jnp.float32)).astype(jnp.int32)
    mmax = jnp.where((mmax + 1) * si <= LLIM - 1, mmax + 1, mmax)
    mmax = jnp.where(mmax * si > LLIM - 1, mmax - 1, mmax)
    covered = (f0i > 0) & (mmax >= 1)
    ncmax = mmax * f0i
    mi = jnp.where(f0i == 1, 2, jnp.where(f0i == 2, 1, 0))
    c0 = jnp.maximum(mi, 1) * f0i
    nc0 = (mi + 1) * f0i

    def p3(k, carry):
        kv, kk3, c, nc = carry
        bump = (kk3 >= nc) & (nc <= ncmax)
        c = jnp.where(bump, nc, c)
        nc = jnp.where(bump, nc + f0i, nc)
        d = jnp.abs(kv - c)
        d = jnp.where(covered, d, MARGIN + 1)
        val = jnp.maximum(1.0 - d.astype(jnp.float32) * (0.5 / MARGIN), 0.5)
        in_v[k] = val
        return kv + 1, kk3 + 1, c, nc

    three = jnp.full((LANES,), MARGIN, jnp.int32)
    lax.fori_loop(0, F, p3, (zero, three, c0, nc0), unroll=8)

    pltpu.async_copy(in_v, o_hbm.at[b, 0, pl.ds(0, F), pl.ds(t0, LANES)],
                     sem).wait()


@jax.jit
def _sc_mask(x2):
    kern = pl.kernel(
        _sc_body,
        out_type=jax.ShapeDtypeStruct((B, 1, F, T), jnp.float32),
        mesh=plsc.VectorSubcoreMesh(core_axis_name="c", subcore_axis_name="s"),
        compiler_params=pltpu.CompilerParams(
            use_tc_tiling_on_sc=False, needs_layout_passes=False),
        scratch_types=[
            pltpu.VMEM((F, LANES), jnp.float32),
            pltpu.VMEM((NCH, LANES), jnp.float32),
            pltpu.SemaphoreType.DMA,
        ],
    )
    return kern(x2)


def kernel(x):
    xp = jnp.pad(x.reshape(B, F, T), ((0, 0), (0, FP - F), (0, 0)),
                 constant_values=0.5)
    return _sc_mask(xp)
